# Initial kernel scaffold; baseline (speedup 1.0000x reference)
import jax
import jax.numpy as jnp
from jax.experimental import pallas as pl


def kernel(x, edge_index, W1, b1, W2, b2):
    raise NotImplementedError("write your pallas kernel here")


# speedup vs baseline: 21.2938x; 21.2938x over previous
import jax
import jax.numpy as jnp
from jax import lax
from jax.experimental import pallas as pl
from jax.experimental.pallas import tpu as pltpu
from jax.experimental.pallas import tpu_sc as plsc

N = 10000
E = 320000
D_IN = 128
D_HID = 128
D_OUT = 16
DH2 = D_HID // 2

NC, NS = 2, 16
NW = NC * NS
CHUNK = 125
NCH_W = E // NW // CHUNK
NCH_S = E // NS // CHUNK
N_PAD = 10240
ROWS_PER_TILE = N_PAD // NS
FCHUNK = 128
FLUSH = ROWS_PER_TILE // FCHUNK

_MESH = plsc.VectorSubcoreMesh(
    core_axis_name="c", subcore_axis_name="s", num_cores=NC, num_subcores=NS
)


def _fill(buf, rows, d, value):
    vec = jnp.full((16,), value, jnp.float32)

    def row(r, carry):
        for j in range(d // 16):
            buf[r, pl.ds(j * 16, 16)] = vec
        return carry

    lax.fori_loop(0, rows, row, 0)


def _zero_acc(fbuf, acc, sid):
    for t in range(FLUSH):
        rows = pl.ds(sid * ROWS_PER_TILE + t * FCHUNK, FCHUNK)
        pltpu.sync_copy(fbuf, acc.at[rows])


def _flush_acc(fbuf, acc, out_plane, sid):
    for t in range(FLUSH):
        rows = pl.ds(sid * ROWS_PER_TILE + t * FCHUNK, FCHUNK)
        pltpu.sync_copy(acc.at[rows], fbuf)
        pltpu.sync_copy(fbuf, out_plane.at[rows])


def _hid_body(table, src_idx, dst_idx, out, src_v, dst_v, gbuf, fbuf, acc, sem):
    cid = lax.axis_index("c")
    sid = lax.axis_index("s")

    _fill(fbuf, FCHUNK, DH2, 0.0)
    _zero_acc(fbuf, acc, sid)
    pltpu.sync_copy(src_idx.at[sid], src_v)
    pltpu.sync_copy(dst_idx.at[sid], dst_v)
    plsc.subcore_barrier()

    def step(j, carry):
        pltpu.async_copy(table.at[cid].at[src_v.at[j]], gbuf, sem).wait()
        pltpu.sync_copy(gbuf, acc.at[dst_v.at[j]], add=True)
        return carry

    lax.fori_loop(0, NCH_S, step, 0)

    plsc.subcore_barrier()
    _flush_acc(fbuf, acc, out.at[cid], sid)


_hid_kernel = pl.kernel(
    _hid_body,
    out_type=jax.ShapeDtypeStruct((NC, N_PAD, DH2), jnp.float32),
    mesh=_MESH,
    scratch_types=[
        pltpu.VMEM((NCH_S, CHUNK), jnp.int32),
        pltpu.VMEM((NCH_S, CHUNK), jnp.int32),
        pltpu.VMEM((CHUNK, DH2), jnp.float32),
        pltpu.VMEM((FCHUNK, DH2), jnp.float32),
        pltpu.VMEM_SHARED((N_PAD, DH2), jnp.float32),
        pltpu.SemaphoreType.DMA,
    ],
    compiler_params=pltpu.CompilerParams(use_tc_tiling_on_sc=False),
)


def _out_prop_body(table, src_idx, dst_idx, out, src_v, dst_v, gbuf, fbuf, acc, sem):
    cid = lax.axis_index("c")
    sid = lax.axis_index("s")
    wid = cid * NS + sid

    _fill(fbuf, FCHUNK, D_OUT, 0.0)
    _zero_acc(fbuf, acc, sid)
    pltpu.sync_copy(src_idx.at[wid], src_v)
    pltpu.sync_copy(dst_idx.at[wid], dst_v)
    plsc.subcore_barrier()

    def step(j, carry):
        pltpu.async_copy(table.at[src_v.at[j]], gbuf, sem).wait()
        pltpu.sync_copy(gbuf, acc.at[dst_v.at[j]], add=True)
        return carry

    lax.fori_loop(0, NCH_W, step, 0)

    plsc.subcore_barrier()
    _flush_acc(fbuf, acc, out.at[cid], sid)


_out_prop_kernel = pl.kernel(
    _out_prop_body,
    out_type=jax.ShapeDtypeStruct((NC, N_PAD, D_OUT), jnp.float32),
    mesh=_MESH,
    scratch_types=[
        pltpu.VMEM((NCH_W, CHUNK), jnp.int32),
        pltpu.VMEM((NCH_W, CHUNK), jnp.int32),
        pltpu.VMEM((CHUNK, D_OUT), jnp.float32),
        pltpu.VMEM((FCHUNK, D_OUT), jnp.float32),
        pltpu.VMEM_SHARED((N_PAD, D_OUT), jnp.float32),
        pltpu.SemaphoreType.DMA,
    ],
    compiler_params=pltpu.CompilerParams(use_tc_tiling_on_sc=False),
)


def _deg_body(dst_idx, out, dst_v, zbuf, obuf, acc):
    cid = lax.axis_index("c")
    sid = lax.axis_index("s")
    wid = cid * NS + sid

    _fill(zbuf, FCHUNK, 16, 0.0)
    _fill(obuf, CHUNK, 16, 1.0)
    _zero_acc(zbuf, acc, sid)
    pltpu.sync_copy(dst_idx.at[wid], dst_v)
    plsc.subcore_barrier()

    def step(j, carry):
        pltpu.sync_copy(obuf, acc.at[dst_v.at[j]], add=True)
        return carry

    lax.fori_loop(0, NCH_W, step, 0)

    plsc.subcore_barrier()
    _flush_acc(zbuf, acc, out.at[cid], sid)


_deg_kernel = pl.kernel(
    _deg_body,
    out_type=jax.ShapeDtypeStruct((NC, N_PAD, 16), jnp.float32),
    mesh=_MESH,
    scratch_types=[
        pltpu.VMEM((NCH_W, CHUNK), jnp.int32),
        pltpu.VMEM((FCHUNK, 16), jnp.float32),
        pltpu.VMEM((CHUNK, 16), jnp.float32),
        pltpu.VMEM_SHARED((N_PAD, 16), jnp.float32),
    ],
    compiler_params=pltpu.CompilerParams(use_tc_tiling_on_sc=False),
)


BM = 1000


def _dinv_of(da_ref, db_ref):
    return lax.rsqrt(da_ref[:, :1] + db_ref[:, :1] + 1.0)


def _mm1_body(x_ref, w_ref, da_ref, db_ref, o_ref):
    dinv = _dinv_of(da_ref, db_ref)
    r = jnp.dot(x_ref[...], w_ref[0], preferred_element_type=jnp.float32) * dinv
    o_ref[...] = r[None]


def _mid_body(al_ref, ar_ref, sl_ref, sr_ref, da_ref, db_ref, b1_ref, w2_ref,
              o_ref):
    dinv = _dinv_of(da_ref, db_ref)
    z = jnp.concatenate(
        [al_ref[0] + sl_ref[0], ar_ref[0] + sr_ref[0]], axis=1
    ) * dinv + b1_ref[...]
    h = jnp.maximum(z, 0.0)
    o_ref[...] = jnp.dot(h, w2_ref[...], preferred_element_type=jnp.float32) * dinv


def _final_body(aa_ref, ab_ref, s2_ref, da_ref, db_ref, b2_ref, o_ref):
    dinv = _dinv_of(da_ref, db_ref)
    z = (aa_ref[...] + ab_ref[...] + s2_ref[...]) * dinv + b2_ref[...]
    m = jnp.max(z, axis=1, keepdims=True)
    e = z - m
    o_ref[...] = e - jnp.log(jnp.sum(jnp.exp(e), axis=1, keepdims=True))


def _row_spec(d):
    return pl.BlockSpec((BM, d), lambda i: (i, 0))


def _mm1(x, w1, dega, degb):
    return pl.pallas_call(
        _mm1_body,
        grid=(N // BM, NC),
        in_specs=[
            pl.BlockSpec((BM, D_IN), lambda i, c: (i, 0)),
            pl.BlockSpec((1, D_IN, DH2), lambda i, c: (c, 0, 0)),
            pl.BlockSpec((BM, 16), lambda i, c: (i, 0)),
            pl.BlockSpec((BM, 16), lambda i, c: (i, 0)),
        ],
        out_specs=pl.BlockSpec((1, BM, DH2), lambda i, c: (c, i, 0)),
        out_shape=jax.ShapeDtypeStruct((NC, N, DH2), jnp.float32),
    )(x, w1.reshape(D_IN, NC, DH2).transpose(1, 0, 2), dega, degb)


def _mid(aggs, scaled1, dega, degb, b1, w2):
    def half(arr, c):
        return pl.BlockSpec((1, BM, DH2), lambda i, _c=c: (_c, i, 0))

    return pl.pallas_call(
        _mid_body,
        grid=(N // BM,),
        in_specs=[
            half(aggs, 0),
            half(aggs, 1),
            half(scaled1, 0),
            half(scaled1, 1),
            _row_spec(16),
            _row_spec(16),
            pl.BlockSpec((1, D_HID), lambda i: (0, 0)),
            pl.BlockSpec((D_HID, D_OUT), lambda i: (0, 0)),
        ],
        out_specs=_row_spec(D_OUT),
        out_shape=jax.ShapeDtypeStruct((N, D_OUT), jnp.float32),
    )(aggs, aggs, scaled1, scaled1, dega, degb, b1, w2)


def _final(aa, ab, s2, dega, degb, b2):
    return pl.pallas_call(
        _final_body,
        grid=(N // BM,),
        in_specs=[
            _row_spec(D_OUT),
            _row_spec(D_OUT),
            _row_spec(D_OUT),
            _row_spec(16),
            _row_spec(16),
            pl.BlockSpec((1, D_OUT), lambda i: (0, 0)),
        ],
        out_specs=_row_spec(D_OUT),
        out_shape=jax.ShapeDtypeStruct((N, D_OUT), jnp.float32),
    )(aa, ab, s2, dega, degb, b2)


def kernel(x, edge_index, W1, b1, W2, b2):
    ei = edge_index.astype(jnp.int32)
    src_w = ei[0].reshape(NW, NCH_W, CHUNK)
    dst_w = ei[1].reshape(NW, NCH_W, CHUNK)
    src_s = ei[0].reshape(NS, NCH_S, CHUNK)
    dst_s = ei[1].reshape(NS, NCH_S, CHUNK)

    degs = _deg_kernel(dst_w)
    dega, degb = degs[0, :N], degs[1, :N]

    scaled1 = _mm1(x, W1, dega, degb)
    agg1 = _hid_kernel(scaled1, src_s, dst_s)
    scaled2 = _mid(agg1[:, :N], scaled1, dega, degb,
                   b1.reshape(1, D_HID), W2)
    agg2 = _out_prop_kernel(scaled2, src_w, dst_w)
    return _final(agg2[0, :N], agg2[1, :N], scaled2, dega, degb,
                  b2.reshape(1, D_OUT))


# speedup vs baseline: 24.6281x; 1.1566x over previous
import jax
import jax.numpy as jnp
from jax import lax
from jax.experimental import pallas as pl
from jax.experimental.pallas import tpu as pltpu
from jax.experimental.pallas import tpu_sc as plsc

N = 10000
E = 320000
D_IN = 128
D_HID = 128
D_OUT = 16
DH2 = D_HID // 2

NC, NS = 2, 16
NW = NC * NS
CHUNK = 125
NCH_W = E // NW // CHUNK
NCH_S = E // NS // CHUNK
N_PAD = 10240
ROWS_PER_TILE = N_PAD // NS
FCHUNK = 128
FLUSH = ROWS_PER_TILE // FCHUNK

_MESH = plsc.VectorSubcoreMesh(
    core_axis_name="c", subcore_axis_name="s", num_cores=NC, num_subcores=NS
)


def _fill(buf, rows, d, value):
    vec = jnp.full((16,), value, jnp.float32)

    def row(r, carry):
        for j in range(d // 16):
            buf[r, pl.ds(j * 16, 16)] = vec
        return carry

    lax.fori_loop(0, rows, row, 0)


def _zero_acc(fbuf, acc, sid):
    for t in range(FLUSH):
        rows = pl.ds(sid * ROWS_PER_TILE + t * FCHUNK, FCHUNK)
        pltpu.sync_copy(fbuf, acc.at[rows])


def _flush_acc(fbuf, acc, out_plane, sid):
    for t in range(FLUSH):
        rows = pl.ds(sid * ROWS_PER_TILE + t * FCHUNK, FCHUNK)
        pltpu.sync_copy(acc.at[rows], fbuf)
        pltpu.sync_copy(fbuf, out_plane.at[rows])


def _prop_body(table, src_idx, dst_idx, out, src_v, dst_v, g0, g1, fbuf, acc,
               s0, s1, *, d, nch, split_cols):
    cid = lax.axis_index("c")
    sid = lax.axis_index("s")

    tbl = table.at[cid] if split_cols else table
    widx = sid if split_cols else cid * NS + sid

    _fill(fbuf, FCHUNK, d, 0.0)
    _zero_acc(fbuf, acc, sid)
    pltpu.sync_copy(src_idx.at[widx], src_v)
    pltpu.sync_copy(dst_idx.at[widx], dst_v)
    plsc.subcore_barrier()

    pltpu.async_copy(tbl.at[src_v.at[0]], g0, s0)

    def grp(g, carry):
        j0 = 2 * g
        pltpu.make_async_copy(tbl.at[src_v.at[j0]], g0, s0).wait()
        pltpu.async_copy(tbl.at[src_v.at[j0 + 1]], g1, s1)
        pltpu.sync_copy(g0, acc.at[dst_v.at[j0]], add=True)
        pltpu.make_async_copy(tbl.at[src_v.at[j0 + 1]], g1, s1).wait()
        jn = jnp.minimum(j0 + 2, nch - 1)
        pltpu.async_copy(tbl.at[src_v.at[jn]], g0, s0)
        pltpu.sync_copy(g1, acc.at[dst_v.at[j0 + 1]], add=True)
        return carry

    lax.fori_loop(0, nch // 2, grp, 0)
    pltpu.make_async_copy(tbl.at[src_v.at[nch - 1]], g0, s0).wait()

    plsc.subcore_barrier()
    _flush_acc(fbuf, acc, out.at[cid], sid)


def _make_prop(d, nch, split_cols):
    import functools

    body = functools.partial(_prop_body, d=d, nch=nch, split_cols=split_cols)
    table_rank_note = None
    del table_rank_note
    return pl.kernel(
        body,
        out_type=jax.ShapeDtypeStruct((NC, N_PAD, d), jnp.float32),
        mesh=_MESH,
        scratch_types=[
            pltpu.VMEM((nch, CHUNK), jnp.int32),
            pltpu.VMEM((nch, CHUNK), jnp.int32),
            pltpu.VMEM((CHUNK, d), jnp.float32),
            pltpu.VMEM((CHUNK, d), jnp.float32),
            pltpu.VMEM((FCHUNK, d), jnp.float32),
            pltpu.VMEM_SHARED((N_PAD, d), jnp.float32),
            pltpu.SemaphoreType.DMA,
            pltpu.SemaphoreType.DMA,
        ],
        compiler_params=pltpu.CompilerParams(use_tc_tiling_on_sc=False),
    )


_hid_kernel = _make_prop(DH2, NCH_S, split_cols=True)
_out_prop_kernel = _make_prop(D_OUT, NCH_W, split_cols=False)


def _deg_body(dst_idx, out, dst_v, zbuf, obuf, acc):
    cid = lax.axis_index("c")
    sid = lax.axis_index("s")
    wid = cid * NS + sid

    _fill(zbuf, FCHUNK, 16, 0.0)
    _fill(obuf, CHUNK, 16, 1.0)
    _zero_acc(zbuf, acc, sid)
    pltpu.sync_copy(dst_idx.at[wid], dst_v)
    plsc.subcore_barrier()

    def step(j, carry):
        pltpu.sync_copy(obuf, acc.at[dst_v.at[j]], add=True)
        return carry

    lax.fori_loop(0, NCH_W, step, 0)

    plsc.subcore_barrier()
    _flush_acc(zbuf, acc, out.at[cid], sid)


_deg_kernel = pl.kernel(
    _deg_body,
    out_type=jax.ShapeDtypeStruct((NC, N_PAD, 16), jnp.float32),
    mesh=_MESH,
    scratch_types=[
        pltpu.VMEM((NCH_W, CHUNK), jnp.int32),
        pltpu.VMEM((FCHUNK, 16), jnp.float32),
        pltpu.VMEM((CHUNK, 16), jnp.float32),
        pltpu.VMEM_SHARED((N_PAD, 16), jnp.float32),
    ],
    compiler_params=pltpu.CompilerParams(use_tc_tiling_on_sc=False),
)


BM = 1000


def _dinv_of(da_ref, db_ref):
    return lax.rsqrt(da_ref[:, :1] + db_ref[:, :1] + 1.0)


def _mm1_body(x_ref, w_ref, da_ref, db_ref, o_ref):
    dinv = _dinv_of(da_ref, db_ref)
    r = jnp.dot(x_ref[...], w_ref[0], preferred_element_type=jnp.float32) * dinv
    o_ref[...] = r[None]


def _mid_body(al_ref, ar_ref, sl_ref, sr_ref, da_ref, db_ref, b1_ref, w2_ref,
              o_ref):
    dinv = _dinv_of(da_ref, db_ref)
    z = jnp.concatenate(
        [al_ref[0] + sl_ref[0], ar_ref[0] + sr_ref[0]], axis=1
    ) * dinv + b1_ref[...]
    h = jnp.maximum(z, 0.0)
    o_ref[...] = jnp.dot(h, w2_ref[...], preferred_element_type=jnp.float32) * dinv


def _final_body(aa_ref, ab_ref, s2_ref, da_ref, db_ref, b2_ref, o_ref):
    dinv = _dinv_of(da_ref, db_ref)
    z = (aa_ref[...] + ab_ref[...] + s2_ref[...]) * dinv + b2_ref[...]
    m = jnp.max(z, axis=1, keepdims=True)
    e = z - m
    o_ref[...] = e - jnp.log(jnp.sum(jnp.exp(e), axis=1, keepdims=True))


def _row_spec(d):
    return pl.BlockSpec((BM, d), lambda i: (i, 0))


def _mm1(x, w1, dega, degb):
    return pl.pallas_call(
        _mm1_body,
        grid=(N // BM, NC),
        in_specs=[
            pl.BlockSpec((BM, D_IN), lambda i, c: (i, 0)),
            pl.BlockSpec((1, D_IN, DH2), lambda i, c: (c, 0, 0)),
            pl.BlockSpec((BM, 16), lambda i, c: (i, 0)),
            pl.BlockSpec((BM, 16), lambda i, c: (i, 0)),
        ],
        out_specs=pl.BlockSpec((1, BM, DH2), lambda i, c: (c, i, 0)),
        out_shape=jax.ShapeDtypeStruct((NC, N, DH2), jnp.float32),
    )(x, w1.reshape(D_IN, NC, DH2).transpose(1, 0, 2), dega, degb)


def _mid(aggs, scaled1, dega, degb, b1, w2):
    def half(arr, c):
        return pl.BlockSpec((1, BM, DH2), lambda i, _c=c: (_c, i, 0))

    return pl.pallas_call(
        _mid_body,
        grid=(N // BM,),
        in_specs=[
            half(aggs, 0),
            half(aggs, 1),
            half(scaled1, 0),
            half(scaled1, 1),
            _row_spec(16),
            _row_spec(16),
            pl.BlockSpec((1, D_HID), lambda i: (0, 0)),
            pl.BlockSpec((D_HID, D_OUT), lambda i: (0, 0)),
        ],
        out_specs=_row_spec(D_OUT),
        out_shape=jax.ShapeDtypeStruct((N, D_OUT), jnp.float32),
    )(aggs, aggs, scaled1, scaled1, dega, degb, b1, w2)


def _final(aa, ab, s2, dega, degb, b2):
    return pl.pallas_call(
        _final_body,
        grid=(N // BM,),
        in_specs=[
            _row_spec(D_OUT),
            _row_spec(D_OUT),
            _row_spec(D_OUT),
            _row_spec(16),
            _row_spec(16),
            pl.BlockSpec((1, D_OUT), lambda i: (0, 0)),
        ],
        out_specs=_row_spec(D_OUT),
        out_shape=jax.ShapeDtypeStruct((N, D_OUT), jnp.float32),
    )(aa, ab, s2, dega, degb, b2)


def kernel(x, edge_index, W1, b1, W2, b2):
    ei = edge_index.astype(jnp.int32)
    src_w = ei[0].reshape(NW, NCH_W, CHUNK)
    dst_w = ei[1].reshape(NW, NCH_W, CHUNK)
    src_s = ei[0].reshape(NS, NCH_S, CHUNK)
    dst_s = ei[1].reshape(NS, NCH_S, CHUNK)

    degs = _deg_kernel(dst_w)
    dega, degb = degs[0, :N], degs[1, :N]

    scaled1 = _mm1(x, W1, dega, degb)
    agg1 = _hid_kernel(scaled1, src_s, dst_s)
    scaled2 = _mid(agg1[:, :N], scaled1, dega, degb,
                   b1.reshape(1, D_HID), W2)
    agg2 = _out_prop_kernel(scaled2, src_w, dst_w)
    return _final(agg2[0, :N], agg2[1, :N], scaled2, dega, degb,
                  b2.reshape(1, D_OUT))


# speedup vs baseline: 33.3990x; 1.3561x over previous
import jax
import jax.numpy as jnp
from jax import lax
from jax.experimental import pallas as pl
from jax.experimental.pallas import tpu as pltpu
from jax.experimental.pallas import tpu_sc as plsc

N = 10000
E = 320000
D_IN = 128
D_HID = 128
D_OUT = 16
DH2 = D_HID // 2

NC, NS = 2, 16
NW = NC * NS
CHUNK = 125
NCH_W = E // NW // CHUNK
NCH_S = E // NS // CHUNK
N_PAD = 10240
ROWS_PER_TILE = N_PAD // NS
FCHUNK = 128
FLUSH = ROWS_PER_TILE // FCHUNK

_MESH = plsc.VectorSubcoreMesh(
    core_axis_name="c", subcore_axis_name="s", num_cores=NC, num_subcores=NS
)


def _fill(buf, rows, d, value):
    vec = jnp.full((16,), value, jnp.float32)

    def row(r, carry):
        for j in range(d // 16):
            buf[r, pl.ds(j * 16, 16)] = vec
        return carry

    lax.fori_loop(0, rows, row, 0)


def _zero_acc(fbuf, acc, sid):
    for t in range(FLUSH):
        rows = pl.ds(sid * ROWS_PER_TILE + t * FCHUNK, FCHUNK)
        pltpu.sync_copy(fbuf, acc.at[rows])


def _flush_acc(fbuf, acc, out_plane, sid):
    for t in range(FLUSH):
        rows = pl.ds(sid * ROWS_PER_TILE + t * FCHUNK, FCHUNK)
        pltpu.sync_copy(acc.at[rows], fbuf)
        pltpu.sync_copy(fbuf, out_plane.at[rows])


NBUF = 4


def _prop_body(table, src_idx, dst_idx, out, src_v, dst_v, gb0, gb1, gb2, gb3,
               fbuf, acc, sm0, sm1, sm2, sm3, *, d, nch, split_cols):
    cid = lax.axis_index("c")
    sid = lax.axis_index("s")
    gb = [gb0, gb1, gb2, gb3]
    sm = [sm0, sm1, sm2, sm3]

    tbl = table.at[cid] if split_cols else table
    widx = sid if split_cols else cid * NS + sid

    _fill(fbuf, FCHUNK, d, 0.0)
    _zero_acc(fbuf, acc, sid)
    pltpu.sync_copy(src_idx.at[widx], src_v)
    pltpu.sync_copy(dst_idx.at[widx], dst_v)
    plsc.subcore_barrier()

    for b in range(NBUF - 1):
        pltpu.async_copy(tbl.at[src_v.at[b]], gb[b], sm[b])

    def grp(q, carry):
        for b in range(NBUF):
            j = NBUF * q + b
            pltpu.make_async_copy(tbl.at[src_v.at[j]], gb[b], sm[b]).wait()
            jn = jnp.minimum(j + NBUF - 1, nch - 1)
            bn = (b + NBUF - 1) % NBUF
            pltpu.async_copy(tbl.at[src_v.at[jn]], gb[bn], sm[bn])
            pltpu.sync_copy(gb[b], acc.at[dst_v.at[j]], add=True)
        return carry

    lax.fori_loop(0, nch // NBUF, grp, 0)
    for b in range(NBUF - 1):
        pltpu.make_async_copy(tbl.at[src_v.at[nch - 1]], gb[b], sm[b]).wait()

    plsc.subcore_barrier()
    _flush_acc(fbuf, acc, out.at[cid], sid)


def _make_prop(d, nch, split_cols):
    import functools

    body = functools.partial(_prop_body, d=d, nch=nch, split_cols=split_cols)
    return pl.kernel(
        body,
        out_type=jax.ShapeDtypeStruct((NC, N_PAD, d), jnp.float32),
        mesh=_MESH,
        scratch_types=[
            pltpu.VMEM((nch, CHUNK), jnp.int32),
            pltpu.VMEM((nch, CHUNK), jnp.int32),
        ]
        + [pltpu.VMEM((CHUNK, d), jnp.float32) for _ in range(NBUF)]
        + [
            pltpu.VMEM((FCHUNK, d), jnp.float32),
            pltpu.VMEM_SHARED((N_PAD, d), jnp.float32),
        ]
        + [pltpu.SemaphoreType.DMA for _ in range(NBUF)],
        compiler_params=pltpu.CompilerParams(use_tc_tiling_on_sc=False),
    )


_hid_kernel = _make_prop(DH2, NCH_S, split_cols=True)
_out_prop_kernel = _make_prop(D_OUT, NCH_W, split_cols=False)


def _deg_body(dst_idx, out, dst_v, zbuf, obuf, acc):
    cid = lax.axis_index("c")
    sid = lax.axis_index("s")
    wid = cid * NS + sid

    _fill(zbuf, FCHUNK, 16, 0.0)
    _fill(obuf, CHUNK, 16, 1.0)
    _zero_acc(zbuf, acc, sid)
    pltpu.sync_copy(dst_idx.at[wid], dst_v)
    plsc.subcore_barrier()

    def step(j, carry):
        pltpu.sync_copy(obuf, acc.at[dst_v.at[j]], add=True)
        return carry

    lax.fori_loop(0, NCH_W, step, 0)

    plsc.subcore_barrier()
    _flush_acc(zbuf, acc, out.at[cid], sid)


_deg_kernel = pl.kernel(
    _deg_body,
    out_type=jax.ShapeDtypeStruct((NC, N_PAD, 16), jnp.float32),
    mesh=_MESH,
    scratch_types=[
        pltpu.VMEM((NCH_W, CHUNK), jnp.int32),
        pltpu.VMEM((FCHUNK, 16), jnp.float32),
        pltpu.VMEM((CHUNK, 16), jnp.float32),
        pltpu.VMEM_SHARED((N_PAD, 16), jnp.float32),
    ],
    compiler_params=pltpu.CompilerParams(use_tc_tiling_on_sc=False),
)


BM = 1000


def _dinv_of(da_ref, db_ref):
    return lax.rsqrt(da_ref[:, :1] + db_ref[:, :1] + 1.0)


def _mm1_body(x_ref, w_ref, o_ref):
    r = jnp.dot(x_ref[...], w_ref[0], preferred_element_type=jnp.float32)
    o_ref[...] = r[None]


def _scale1_body(xw_ref, da_ref, db_ref, o_ref):
    dinv = _dinv_of(da_ref, db_ref)
    o_ref[...] = (xw_ref[0] * dinv)[None]


def _mid_body(al_ref, ar_ref, sl_ref, sr_ref, da_ref, db_ref, b1_ref, w2_ref,
              o_ref):
    dinv = _dinv_of(da_ref, db_ref)
    z = jnp.concatenate(
        [al_ref[0] + sl_ref[0], ar_ref[0] + sr_ref[0]], axis=1
    ) * dinv + b1_ref[...]
    h = jnp.maximum(z, 0.0)
    o_ref[...] = jnp.dot(h, w2_ref[...], preferred_element_type=jnp.float32) * dinv


def _final_body(aa_ref, ab_ref, s2_ref, da_ref, db_ref, b2_ref, o_ref):
    dinv = _dinv_of(da_ref, db_ref)
    z = (aa_ref[...] + ab_ref[...] + s2_ref[...]) * dinv + b2_ref[...]
    m = jnp.max(z, axis=1, keepdims=True)
    e = z - m
    o_ref[...] = e - jnp.log(jnp.sum(jnp.exp(e), axis=1, keepdims=True))


def _row_spec(d):
    return pl.BlockSpec((BM, d), lambda i: (i, 0))


def _mm1(x, w1):
    return pl.pallas_call(
        _mm1_body,
        grid=(N // BM, NC),
        in_specs=[
            pl.BlockSpec((BM, D_IN), lambda i, c: (i, 0)),
            pl.BlockSpec((1, D_IN, DH2), lambda i, c: (c, 0, 0)),
        ],
        out_specs=pl.BlockSpec((1, BM, DH2), lambda i, c: (c, i, 0)),
        out_shape=jax.ShapeDtypeStruct((NC, N, DH2), jnp.float32),
    )(x, w1.reshape(D_IN, NC, DH2).transpose(1, 0, 2))


def _scale1(xw, dega, degb):
    return pl.pallas_call(
        _scale1_body,
        grid=(N // BM, NC),
        in_specs=[
            pl.BlockSpec((1, BM, DH2), lambda i, c: (c, i, 0)),
            pl.BlockSpec((BM, 16), lambda i, c: (i, 0)),
            pl.BlockSpec((BM, 16), lambda i, c: (i, 0)),
        ],
        out_specs=pl.BlockSpec((1, BM, DH2), lambda i, c: (c, i, 0)),
        out_shape=jax.ShapeDtypeStruct((NC, N, DH2), jnp.float32),
    )(xw, dega, degb)


def _mid(aggs, scaled1, dega, degb, b1, w2):
    def half(arr, c):
        return pl.BlockSpec((1, BM, DH2), lambda i, _c=c: (_c, i, 0))

    return pl.pallas_call(
        _mid_body,
        grid=(N // BM,),
        in_specs=[
            half(aggs, 0),
            half(aggs, 1),
            half(scaled1, 0),
            half(scaled1, 1),
            _row_spec(16),
            _row_spec(16),
            pl.BlockSpec((1, D_HID), lambda i: (0, 0)),
            pl.BlockSpec((D_HID, D_OUT), lambda i: (0, 0)),
        ],
        out_specs=_row_spec(D_OUT),
        out_shape=jax.ShapeDtypeStruct((N, D_OUT), jnp.float32),
    )(aggs, aggs, scaled1, scaled1, dega, degb, b1, w2)


def _final(aa, ab, s2, dega, degb, b2):
    return pl.pallas_call(
        _final_body,
        grid=(N // BM,),
        in_specs=[
            _row_spec(D_OUT),
            _row_spec(D_OUT),
            _row_spec(D_OUT),
            _row_spec(16),
            _row_spec(16),
            pl.BlockSpec((1, D_OUT), lambda i: (0, 0)),
        ],
        out_specs=_row_spec(D_OUT),
        out_shape=jax.ShapeDtypeStruct((N, D_OUT), jnp.float32),
    )(aa, ab, s2, dega, degb, b2)


def kernel(x, edge_index, W1, b1, W2, b2):
    ei = edge_index.astype(jnp.int32)
    src_w = ei[0].reshape(NW, NCH_W, CHUNK)
    dst_w = ei[1].reshape(NW, NCH_W, CHUNK)
    src_s = ei[0].reshape(NS, NCH_S, CHUNK)
    dst_s = ei[1].reshape(NS, NCH_S, CHUNK)

    degs = _deg_kernel(dst_w)
    dega, degb = degs[0, :N], degs[1, :N]

    xw = _mm1(x, W1)
    scaled1 = _scale1(xw, dega, degb)
    agg1 = _hid_kernel(scaled1, src_s, dst_s)
    scaled2 = _mid(agg1[:, :N], scaled1, dega, degb,
                   b1.reshape(1, D_HID), W2)
    agg2 = _out_prop_kernel(scaled2, src_w, dst_w)
    return _final(agg2[0, :N], agg2[1, :N], scaled2, dega, degb,
                  b2.reshape(1, D_OUT))


# speedup vs baseline: 40.8367x; 1.2227x over previous
import functools

import jax
import jax.numpy as jnp
from jax import lax
from jax.experimental import pallas as pl
from jax.experimental.pallas import tpu as pltpu
from jax.experimental.pallas import tpu_sc as plsc

N = 10000
E = 320000
D_IN = 128
D_HID = 128
D_OUT = 16
D_DEG = 16
DH2 = D_HID // 2

NC, NS = 2, 16
NW = NC * NS
EPW = E // NW
CHUNK = 80
NCH = EPW // CHUNK
NCH2 = (E // NS) // CHUNK
N_PAD = 10240
ROWS_PER_TILE = N_PAD // NS
FCHUNK = 128
FLUSH = ROWS_PER_TILE // FCHUNK
NBUF = 5

_MESH = plsc.VectorSubcoreMesh(
    core_axis_name="c", subcore_axis_name="s", num_cores=NC, num_subcores=NS
)


def _fill(buf, rows, d, value):
    vec = jnp.full((16,), value, jnp.float32)

    def row(r, carry):
        for j in range(d // 16):
            buf[r, pl.ds(j * 16, 16)] = vec
        return carry

    lax.fori_loop(0, rows, row, 0)


def _zero_acc(fbuf, acc, sid):
    for t in range(FLUSH):
        rows = pl.ds(sid * ROWS_PER_TILE + t * FCHUNK, FCHUNK)
        pltpu.sync_copy(fbuf, acc.at[rows])


def _flush_acc(fbuf, acc, out_plane, sid):
    for t in range(FLUSH):
        rows = pl.ds(sid * ROWS_PER_TILE + t * FCHUNK, FCHUNK)
        pltpu.sync_copy(acc.at[rows], fbuf)
        pltpu.sync_copy(fbuf, out_plane.at[rows])


def _prop_body(table, src_idx, dst_idx, out, src_v, dst_v, gb0, gb1, gb2, gb3,
               gb4, fbuf, acc, sm0, sm1, sm2, sm3, sm4, *, d, col_split, nch):
    cid = lax.axis_index("c")
    sid = lax.axis_index("s")
    wid = cid * NS + sid
    gb = [gb0, gb1, gb2, gb3, gb4]
    sm = [sm0, sm1, sm2, sm3, sm4]

    _fill(fbuf, FCHUNK, d, 0.0)
    _zero_acc(fbuf, acc, sid)
    pltpu.sync_copy(src_idx.at[wid], src_v)
    if col_split:
        pltpu.sync_copy(dst_idx.at[sid], dst_v)
    else:
        pltpu.sync_copy(dst_idx.at[wid], dst_v)
    plsc.subcore_barrier()

    def gslice(j):
        return table.at[src_v.at[j]]

    for b in range(NBUF - 1):
        pltpu.async_copy(gslice(b), gb[b], sm[b])

    def grp(q, carry):
        for b in range(NBUF):
            j = NBUF * q + b
            pltpu.make_async_copy(gslice(j), gb[b], sm[b]).wait()
            jn = jnp.minimum(j + NBUF - 1, nch - 1)
            bn = (b + NBUF - 1) % NBUF
            pltpu.async_copy(gslice(jn), gb[bn], sm[bn])
            pltpu.sync_copy(gb[b], acc.at[dst_v.at[j]], add=True)
        return carry

    lax.fori_loop(0, nch // NBUF, grp, 0)
    for b in range(NBUF - 1):
        pltpu.make_async_copy(gslice(nch - 1), gb[b], sm[b]).wait()

    plsc.subcore_barrier()
    _flush_acc(fbuf, acc, out.at[cid], sid)


def _make_prop(d, col_split, nch):
    return pl.kernel(
        functools.partial(_prop_body, d=d, col_split=col_split, nch=nch),
        out_type=jax.ShapeDtypeStruct((NC, N_PAD, d), jnp.float32),
        mesh=_MESH,
        scratch_types=[
            pltpu.VMEM((nch, CHUNK), jnp.int32),
            pltpu.VMEM((nch, CHUNK), jnp.int32),
        ]
        + [pltpu.VMEM((CHUNK, d), jnp.float32) for _ in range(NBUF)]
        + [
            pltpu.VMEM((FCHUNK, d), jnp.float32),
            pltpu.VMEM_SHARED((N_PAD, d), jnp.float32),
        ]
        + [pltpu.SemaphoreType.DMA for _ in range(NBUF)],
        compiler_params=pltpu.CompilerParams(use_tc_tiling_on_sc=False),
    )


_hid_kernel = _make_prop(DH2, col_split=True, nch=NCH2)
_out_prop_kernel = _make_prop(D_OUT, col_split=False, nch=NCH)


def _deg_body(dst_idx, out, dst_v, zbuf, obuf, acc):
    cid = lax.axis_index("c")
    sid = lax.axis_index("s")
    wid = cid * NS + sid

    _fill(zbuf, FCHUNK, D_DEG, 0.0)
    _fill(obuf, CHUNK, D_DEG, 1.0)
    _zero_acc(zbuf, acc, sid)
    pltpu.sync_copy(dst_idx.at[wid], dst_v)
    plsc.subcore_barrier()

    def step(j, carry):
        pltpu.sync_copy(obuf, acc.at[dst_v.at[j]], add=True)
        return carry

    lax.fori_loop(0, NCH, step, 0)

    plsc.subcore_barrier()
    _flush_acc(zbuf, acc, out.at[cid], sid)


_deg_kernel = pl.kernel(
    _deg_body,
    out_type=jax.ShapeDtypeStruct((NC, N_PAD, D_DEG), jnp.float32),
    mesh=_MESH,
    scratch_types=[
        pltpu.VMEM((NCH, CHUNK), jnp.int32),
        pltpu.VMEM((FCHUNK, D_DEG), jnp.float32),
        pltpu.VMEM((CHUNK, D_DEG), jnp.float32),
        pltpu.VMEM_SHARED((N_PAD, D_DEG), jnp.float32),
    ],
    compiler_params=pltpu.CompilerParams(use_tc_tiling_on_sc=False),
)


BM = 2000


def _dinv_of(da_ref, db_ref):
    return lax.rsqrt(da_ref[0, :, :1] + db_ref[0, :, :1] + 1.0)


def _mm1_body(x_ref, w_ref, o_ref):
    o_ref[...] = jnp.dot(x_ref[...], w_ref[...],
                         preferred_element_type=jnp.float32)


def _scale1_body(xw_ref, da_ref, db_ref, o_ref):
    o_ref[...] = xw_ref[...] * _dinv_of(da_ref, db_ref)


def _mid_body(aa_ref, ab_ref, s1_ref, da_ref, db_ref, b1_ref, w2_ref, o_ref):
    dinv = _dinv_of(da_ref, db_ref)
    agg = jnp.concatenate([aa_ref[0], ab_ref[0]], axis=1)
    z = (agg + s1_ref[...]) * dinv + b1_ref[...]
    h = jnp.maximum(z, 0.0)
    o_ref[...] = jnp.dot(h, w2_ref[...], preferred_element_type=jnp.float32) * dinv


def _final_body(aa_ref, ab_ref, s2_ref, da_ref, db_ref, b2_ref, o_ref):
    dinv = _dinv_of(da_ref, db_ref)
    z = (aa_ref[0] + ab_ref[0] + s2_ref[...]) * dinv + b2_ref[...]
    m = jnp.max(z, axis=1, keepdims=True)
    e = z - m
    o_ref[...] = e - jnp.log(jnp.sum(jnp.exp(e), axis=1, keepdims=True))


def _row_spec(d):
    return pl.BlockSpec((BM, d), lambda i: (i, 0))


def _plane_spec(d, c):
    return pl.BlockSpec((1, BM, d), lambda i, _c=c: (_c, i, 0))


def _deg_specs():
    return [_plane_spec(D_DEG, 0), _plane_spec(D_DEG, 1)]


def _mm1(x, w1):
    return pl.pallas_call(
        _mm1_body,
        grid=(N // BM,),
        in_specs=[
            _row_spec(D_IN),
            pl.BlockSpec((D_IN, D_HID), lambda i: (0, 0)),
        ],
        out_specs=_row_spec(D_HID),
        out_shape=jax.ShapeDtypeStruct((N, D_HID), jnp.float32),
    )(x, w1)


def _scale1(xw, degs):
    return pl.pallas_call(
        _scale1_body,
        grid=(N // BM,),
        in_specs=[_row_spec(D_HID)] + _deg_specs(),
        out_specs=_row_spec(D_HID),
        out_shape=jax.ShapeDtypeStruct((N, D_HID), jnp.float32),
    )(xw, degs, degs)


def _mid(agg1, scaled1, degs, b1, w2):
    return pl.pallas_call(
        _mid_body,
        grid=(N // BM,),
        in_specs=[
            _plane_spec(DH2, 0),
            _plane_spec(DH2, 1),
            _row_spec(D_HID),
        ]
        + _deg_specs()
        + [
            pl.BlockSpec((1, D_HID), lambda i: (0, 0)),
            pl.BlockSpec((D_HID, D_OUT), lambda i: (0, 0)),
        ],
        out_specs=_row_spec(D_OUT),
        out_shape=jax.ShapeDtypeStruct((N, D_OUT), jnp.float32),
    )(agg1, agg1, scaled1, degs, degs, b1, w2)


def _final(agg2, scaled2, degs, b2):
    return pl.pallas_call(
        _final_body,
        grid=(N // BM,),
        in_specs=[
            _plane_spec(D_OUT, 0),
            _plane_spec(D_OUT, 1),
            _row_spec(D_OUT),
        ]
        + _deg_specs()
        + [pl.BlockSpec((1, D_OUT), lambda i: (0, 0))],
        out_specs=_row_spec(D_OUT),
        out_shape=jax.ShapeDtypeStruct((N, D_OUT), jnp.float32),
    )(agg2, agg2, scaled2, degs, degs, b2)


def kernel(x, edge_index, W1, b1, W2, b2):
    ei = edge_index.astype(jnp.int32)
    src = ei[0].reshape(NW, NCH, CHUNK)
    dst = ei[1].reshape(NW, NCH, CHUNK)

    src16 = ei[0].reshape(NS, NCH2, CHUNK) * 2
    src_cols = jnp.concatenate([src16, src16 + 1])
    dst16 = ei[1].reshape(NS, NCH2, CHUNK)

    degs = _deg_kernel(dst)
    xw = _mm1(x, W1)
    scaled1 = _scale1(xw, degs)
    table1 = scaled1.reshape(NC * N, DH2)
    agg1 = _hid_kernel(table1, src_cols, dst16)
    scaled2 = _mid(agg1, scaled1, degs,
                   b1.reshape(1, D_HID), W2)
    agg2 = _out_prop_kernel(scaled2, src, dst)
    return _final(agg2, scaled2, degs, b2.reshape(1, D_OUT))


# speedup vs baseline: 41.3354x; 1.0122x over previous
import functools

import jax
import jax.numpy as jnp
from jax import lax
from jax.experimental import pallas as pl
from jax.experimental.pallas import tpu as pltpu
from jax.experimental.pallas import tpu_sc as plsc

N = 10000
E = 320000
D_IN = 128
D_HID = 128
D_OUT = 16
D_DEG = 16
DH2 = D_HID // 2

NC, NS = 2, 16
NW = NC * NS
EPW = E // NW
CHUNK = 80
NCH = EPW // CHUNK
CHUNK2 = 80
NCH2 = (E // NS) // CHUNK2
N_PAD = 10240
ROWS_PER_TILE = N_PAD // NS
FCHUNK = 128
FLUSH = ROWS_PER_TILE // FCHUNK
NBUF = 5

_MESH = plsc.VectorSubcoreMesh(
    core_axis_name="c", subcore_axis_name="s", num_cores=NC, num_subcores=NS
)


def _fill(buf, rows, d, value):
    vec = jnp.full((16,), value, jnp.float32)

    def row(r, carry):
        for j in range(d // 16):
            buf[r, pl.ds(j * 16, 16)] = vec
        return carry

    lax.fori_loop(0, rows, row, 0)


def _zero_acc(fbuf, acc, sid):
    for t in range(FLUSH):
        rows = pl.ds(sid * ROWS_PER_TILE + t * FCHUNK, FCHUNK)
        pltpu.sync_copy(fbuf, acc.at[rows])


def _flush_acc(fbuf, acc, out_plane, sid):
    for t in range(FLUSH):
        rows = pl.ds(sid * ROWS_PER_TILE + t * FCHUNK, FCHUNK)
        pltpu.sync_copy(acc.at[rows], fbuf)
        pltpu.sync_copy(fbuf, out_plane.at[rows])


def _prop_body(table, src_idx, dst_idx, out, src_v, dst_v, gb0, gb1, gb2, gb3,
               gb4, fbuf, acc, sm0, sm1, sm2, sm3, sm4, *, d, col_split, nch):
    cid = lax.axis_index("c")
    sid = lax.axis_index("s")
    wid = cid * NS + sid
    gb = [gb0, gb1, gb2, gb3, gb4]
    sm = [sm0, sm1, sm2, sm3, sm4]

    _fill(fbuf, FCHUNK, d, 0.0)
    _zero_acc(fbuf, acc, sid)
    pltpu.sync_copy(src_idx.at[wid], src_v)
    if col_split:
        pltpu.sync_copy(dst_idx.at[sid], dst_v)
    else:
        pltpu.sync_copy(dst_idx.at[wid], dst_v)
    plsc.subcore_barrier()

    def gslice(j):
        return table.at[src_v.at[j]]

    for b in range(NBUF - 1):
        pltpu.async_copy(gslice(b), gb[b], sm[b])

    def grp(q, carry):
        for b in range(NBUF):
            j = NBUF * q + b
            pltpu.make_async_copy(gslice(j), gb[b], sm[b]).wait()
            jn = jnp.minimum(j + NBUF - 1, nch - 1)
            bn = (b + NBUF - 1) % NBUF
            pltpu.async_copy(gslice(jn), gb[bn], sm[bn])
            pltpu.sync_copy(gb[b], acc.at[dst_v.at[j]], add=True)
        return carry

    lax.fori_loop(0, nch // NBUF, grp, 0)
    for b in range(NBUF - 1):
        pltpu.make_async_copy(gslice(nch - 1), gb[b], sm[b]).wait()

    plsc.subcore_barrier()
    _flush_acc(fbuf, acc, out.at[cid], sid)


def _make_prop(d, col_split, nch, chunk):
    return pl.kernel(
        functools.partial(_prop_body, d=d, col_split=col_split, nch=nch),
        out_type=jax.ShapeDtypeStruct((NC, N_PAD, d), jnp.float32),
        mesh=_MESH,
        scratch_types=[
            pltpu.VMEM((nch, chunk), jnp.int32),
            pltpu.VMEM((nch, chunk), jnp.int32),
        ]
        + [pltpu.VMEM((chunk, d), jnp.float32) for _ in range(NBUF)]
        + [
            pltpu.VMEM((FCHUNK, d), jnp.float32),
            pltpu.VMEM_SHARED((N_PAD, d), jnp.float32),
        ]
        + [pltpu.SemaphoreType.DMA for _ in range(NBUF)],
        compiler_params=pltpu.CompilerParams(use_tc_tiling_on_sc=False),
    )


_hid_kernel = _make_prop(DH2, col_split=True, nch=NCH2, chunk=CHUNK2)


def _prop_staged_body(table, src_idx, dst_idx, out, src_v, dst_v, gb0, gb1,
                      gb2, gb3, gb4, fbuf, acc, tbl, sm0, sm1, sm2, sm3, sm4,
                      *, d, nch):
    cid = lax.axis_index("c")
    sid = lax.axis_index("s")
    wid = cid * NS + sid
    gb = [gb0, gb1, gb2, gb3, gb4]
    sm = [sm0, sm1, sm2, sm3, sm4]

    _fill(fbuf, FCHUNK, d, 0.0)
    _zero_acc(fbuf, acc, sid)
    for t in range(FLUSH):
        rows = pl.ds(sid * ROWS_PER_TILE + t * FCHUNK, FCHUNK)
        pltpu.sync_copy(table.at[rows], tbl.at[rows])
    pltpu.sync_copy(src_idx.at[wid], src_v)
    pltpu.sync_copy(dst_idx.at[wid], dst_v)
    plsc.subcore_barrier()

    def gslice(j):
        return tbl.at[src_v.at[j]]

    for b in range(NBUF - 1):
        pltpu.async_copy(gslice(b), gb[b], sm[b])

    def grp(q, carry):
        for b in range(NBUF):
            j = NBUF * q + b
            pltpu.make_async_copy(gslice(j), gb[b], sm[b]).wait()
            jn = jnp.minimum(j + NBUF - 1, nch - 1)
            bn = (b + NBUF - 1) % NBUF
            pltpu.async_copy(gslice(jn), gb[bn], sm[bn])
            pltpu.sync_copy(gb[b], acc.at[dst_v.at[j]], add=True)
        return carry

    lax.fori_loop(0, nch // NBUF, grp, 0)
    for b in range(NBUF - 1):
        pltpu.make_async_copy(gslice(nch - 1), gb[b], sm[b]).wait()

    plsc.subcore_barrier()
    _flush_acc(fbuf, acc, out.at[cid], sid)


_out_prop_kernel = pl.kernel(
    functools.partial(_prop_staged_body, d=D_OUT, nch=NCH),
    out_type=jax.ShapeDtypeStruct((NC, N_PAD, D_OUT), jnp.float32),
    mesh=_MESH,
    scratch_types=[
        pltpu.VMEM((NCH, CHUNK), jnp.int32),
        pltpu.VMEM((NCH, CHUNK), jnp.int32),
    ]
    + [pltpu.VMEM((CHUNK, D_OUT), jnp.float32) for _ in range(NBUF)]
    + [
        pltpu.VMEM((FCHUNK, D_OUT), jnp.float32),
        pltpu.VMEM_SHARED((N_PAD, D_OUT), jnp.float32),
        pltpu.VMEM_SHARED((N_PAD, D_OUT), jnp.float32),
    ]
    + [pltpu.SemaphoreType.DMA for _ in range(NBUF)],
    compiler_params=pltpu.CompilerParams(use_tc_tiling_on_sc=False),
)


def _deg_body(dst_idx, out, dst_v, zbuf, obuf, acc):
    cid = lax.axis_index("c")
    sid = lax.axis_index("s")
    wid = cid * NS + sid

    _fill(zbuf, FCHUNK, D_DEG, 0.0)
    _fill(obuf, CHUNK, D_DEG, 1.0)
    _zero_acc(zbuf, acc, sid)
    pltpu.sync_copy(dst_idx.at[wid], dst_v)
    plsc.subcore_barrier()

    def step(j, carry):
        pltpu.sync_copy(obuf, acc.at[dst_v.at[j]], add=True)
        return carry

    lax.fori_loop(0, NCH, step, 0)

    plsc.subcore_barrier()
    _flush_acc(zbuf, acc, out.at[cid], sid)


_deg_kernel = pl.kernel(
    _deg_body,
    out_type=jax.ShapeDtypeStruct((NC, N_PAD, D_DEG), jnp.float32),
    mesh=_MESH,
    scratch_types=[
        pltpu.VMEM((NCH, CHUNK), jnp.int32),
        pltpu.VMEM((FCHUNK, D_DEG), jnp.float32),
        pltpu.VMEM((CHUNK, D_DEG), jnp.float32),
        pltpu.VMEM_SHARED((N_PAD, D_DEG), jnp.float32),
    ],
    compiler_params=pltpu.CompilerParams(use_tc_tiling_on_sc=False),
)


BM = 2000


def _dinv_of(da_ref, db_ref):
    return lax.rsqrt(da_ref[0, :, :1] + db_ref[0, :, :1] + 1.0)


def _mm1_body(x_ref, w_ref, o_ref):
    o_ref[...] = jnp.dot(x_ref[...], w_ref[...],
                         preferred_element_type=jnp.float32)


def _scale1_body(xw_ref, da_ref, db_ref, o_ref):
    o_ref[...] = xw_ref[...] * _dinv_of(da_ref, db_ref)


def _mid_body(aa_ref, ab_ref, s1_ref, da_ref, db_ref, b1_ref, w2_ref, o_ref):
    dinv = _dinv_of(da_ref, db_ref)
    agg = jnp.concatenate([aa_ref[0], ab_ref[0]], axis=1)
    z = (agg + s1_ref[...]) * dinv + b1_ref[...]
    h = jnp.maximum(z, 0.0)
    o_ref[...] = jnp.dot(h, w2_ref[...], preferred_element_type=jnp.float32) * dinv


def _final_body(aa_ref, ab_ref, s2_ref, da_ref, db_ref, b2_ref, o_ref):
    dinv = _dinv_of(da_ref, db_ref)
    z = (aa_ref[0] + ab_ref[0] + s2_ref[...]) * dinv + b2_ref[...]
    m = jnp.max(z, axis=1, keepdims=True)
    e = z - m
    o_ref[...] = e - jnp.log(jnp.sum(jnp.exp(e), axis=1, keepdims=True))


def _row_spec(d):
    return pl.BlockSpec((BM, d), lambda i: (i, 0))


def _plane_spec(d, c):
    return pl.BlockSpec((1, BM, d), lambda i, _c=c: (_c, i, 0))


def _deg_specs():
    return [_plane_spec(D_DEG, 0), _plane_spec(D_DEG, 1)]


def _mm1(x, w1):
    return pl.pallas_call(
        _mm1_body,
        grid=(N // BM,),
        in_specs=[
            _row_spec(D_IN),
            pl.BlockSpec((D_IN, D_HID), lambda i: (0, 0)),
        ],
        out_specs=_row_spec(D_HID),
        out_shape=jax.ShapeDtypeStruct((N, D_HID), jnp.float32),
    )(x, w1)


def _scale1(xw, degs):
    return pl.pallas_call(
        _scale1_body,
        grid=(N // BM,),
        in_specs=[_row_spec(D_HID)] + _deg_specs(),
        out_specs=_row_spec(D_HID),
        out_shape=jax.ShapeDtypeStruct((N, D_HID), jnp.float32),
    )(xw, degs, degs)


def _mid(agg1, scaled1, degs, b1, w2):
    return pl.pallas_call(
        _mid_body,
        grid=(N // BM,),
        in_specs=[
            _plane_spec(DH2, 0),
            _plane_spec(DH2, 1),
            _row_spec(D_HID),
        ]
        + _deg_specs()
        + [
            pl.BlockSpec((1, D_HID), lambda i: (0, 0)),
            pl.BlockSpec((D_HID, D_OUT), lambda i: (0, 0)),
        ],
        out_specs=_row_spec(D_OUT),
        out_shape=jax.ShapeDtypeStruct((N, D_OUT), jnp.float32),
    )(agg1, agg1, scaled1, degs, degs, b1, w2)


def _final(agg2, scaled2, degs, b2):
    return pl.pallas_call(
        _final_body,
        grid=(N // BM,),
        in_specs=[
            _plane_spec(D_OUT, 0),
            _plane_spec(D_OUT, 1),
            _row_spec(D_OUT),
        ]
        + _deg_specs()
        + [pl.BlockSpec((1, D_OUT), lambda i: (0, 0))],
        out_specs=_row_spec(D_OUT),
        out_shape=jax.ShapeDtypeStruct((N, D_OUT), jnp.float32),
    )(agg2, agg2, scaled2, degs, degs, b2)


def kernel(x, edge_index, W1, b1, W2, b2):
    ei = edge_index.astype(jnp.int32)
    src = ei[0].reshape(NW, NCH, CHUNK)
    dst = ei[1].reshape(NW, NCH, CHUNK)

    src16 = ei[0].reshape(NS, NCH2, CHUNK2) * 2
    src_cols = jnp.concatenate([src16, src16 + 1])
    dst16 = ei[1].reshape(NS, NCH2, CHUNK2)

    degs = _deg_kernel(dst)
    xw = _mm1(x, W1)
    scaled1 = _scale1(xw, degs)
    table1 = scaled1.reshape(NC * N, DH2)
    agg1 = _hid_kernel(table1, src_cols, dst16)
    scaled2 = _mid(agg1, scaled1, degs,
                   b1.reshape(1, D_HID), W2)
    scaled2_pad = jnp.concatenate(
        [scaled2, jnp.zeros((N_PAD - N, D_OUT), jnp.float32)])
    agg2 = _out_prop_kernel(scaled2_pad, src, dst)
    return _final(agg2, scaled2, degs, b2.reshape(1, D_OUT))


# speedup vs baseline: 41.3792x; 1.0011x over previous
import functools

import jax
import jax.numpy as jnp
from jax import lax
from jax.experimental import pallas as pl
from jax.experimental.pallas import tpu as pltpu
from jax.experimental.pallas import tpu_sc as plsc

N = 10000
E = 320000
D_IN = 128
D_HID = 128
D_OUT = 16
D_DEG = 16
DH2 = D_HID // 2

NC, NS = 2, 16
NW = NC * NS
EPW = E // NW
CHUNK = 80
NCH = EPW // CHUNK
CHUNK2 = 80
NCH2 = (E // NS) // CHUNK2
N_PAD = 10240
ROWS_PER_TILE = N_PAD // NS
FCHUNK = 128
FLUSH = ROWS_PER_TILE // FCHUNK
NBUF = 5

_MESH = plsc.VectorSubcoreMesh(
    core_axis_name="c", subcore_axis_name="s", num_cores=NC, num_subcores=NS
)


def _fill(buf, rows, d, value):
    vec = jnp.full((16,), value, jnp.float32)

    def row(r, carry):
        for j in range(d // 16):
            buf[r, pl.ds(j * 16, 16)] = vec
        return carry

    lax.fori_loop(0, rows, row, 0)


def _zero_acc(fbuf, acc, sid):
    for t in range(FLUSH):
        rows = pl.ds(sid * ROWS_PER_TILE + t * FCHUNK, FCHUNK)
        pltpu.sync_copy(fbuf, acc.at[rows])


def _flush_acc(fbuf, acc, out_plane, sid):
    for t in range(FLUSH):
        rows = pl.ds(sid * ROWS_PER_TILE + t * FCHUNK, FCHUNK)
        pltpu.sync_copy(acc.at[rows], fbuf)
        pltpu.sync_copy(fbuf, out_plane.at[rows])


def _prop_body(table, src_idx, dst_idx, out, src_v, dst_v, gb0, gb1, gb2, gb3,
               gb4, fbuf, acc, sm0, sm1, sm2, sm3, sm4, *, d, col_split, nch):
    cid = lax.axis_index("c")
    sid = lax.axis_index("s")
    wid = cid * NS + sid
    gb = [gb0, gb1, gb2, gb3, gb4]
    sm = [sm0, sm1, sm2, sm3, sm4]

    _fill(fbuf, FCHUNK, d, 0.0)
    _zero_acc(fbuf, acc, sid)
    pltpu.sync_copy(src_idx.at[wid], src_v)
    if col_split:
        pltpu.sync_copy(dst_idx.at[sid], dst_v)
    else:
        pltpu.sync_copy(dst_idx.at[wid], dst_v)
    plsc.subcore_barrier()

    def gslice(j):
        return table.at[src_v.at[j]]

    for b in range(NBUF - 1):
        pltpu.async_copy(gslice(b), gb[b], sm[b])

    def grp(q, carry):
        for b in range(NBUF):
            j = NBUF * q + b
            pltpu.make_async_copy(gslice(j), gb[b], sm[b]).wait()
            jn = jnp.minimum(j + NBUF - 1, nch - 1)
            bn = (b + NBUF - 1) % NBUF
            pltpu.async_copy(gslice(jn), gb[bn], sm[bn])
            pltpu.sync_copy(gb[b], acc.at[dst_v.at[j]], add=True)
        return carry

    lax.fori_loop(0, nch // NBUF, grp, 0)
    for b in range(NBUF - 1):
        pltpu.make_async_copy(gslice(nch - 1), gb[b], sm[b]).wait()

    plsc.subcore_barrier()
    _flush_acc(fbuf, acc, out.at[cid], sid)


def _make_prop(d, col_split, nch, chunk):
    return pl.kernel(
        functools.partial(_prop_body, d=d, col_split=col_split, nch=nch),
        out_type=jax.ShapeDtypeStruct((NC, N_PAD, d), jnp.float32),
        mesh=_MESH,
        scratch_types=[
            pltpu.VMEM((nch, chunk), jnp.int32),
            pltpu.VMEM((nch, chunk), jnp.int32),
        ]
        + [pltpu.VMEM((chunk, d), jnp.float32) for _ in range(NBUF)]
        + [
            pltpu.VMEM((FCHUNK, d), jnp.float32),
            pltpu.VMEM_SHARED((N_PAD, d), jnp.float32),
        ]
        + [pltpu.SemaphoreType.DMA for _ in range(NBUF)],
        compiler_params=pltpu.CompilerParams(use_tc_tiling_on_sc=False),
    )


_hid_kernel = _make_prop(DH2, col_split=True, nch=NCH2, chunk=CHUNK2)


def _prop_staged_body(table, src_idx, dst_idx, out, src_v, dst_v, gb0, gb1,
                      gb2, gb3, gb4, fbuf, acc, tbl, sm0, sm1, sm2, sm3, sm4,
                      *, d, nch):
    cid = lax.axis_index("c")
    sid = lax.axis_index("s")
    wid = cid * NS + sid
    gb = [gb0, gb1, gb2, gb3, gb4]
    sm = [sm0, sm1, sm2, sm3, sm4]

    _fill(fbuf, FCHUNK, d, 0.0)
    _zero_acc(fbuf, acc, sid)
    for t in range(FLUSH):
        rows = pl.ds(sid * ROWS_PER_TILE + t * FCHUNK, FCHUNK)
        pltpu.sync_copy(table.at[rows], tbl.at[rows])
    pltpu.sync_copy(src_idx.at[wid], src_v)
    pltpu.sync_copy(dst_idx.at[wid], dst_v)
    plsc.subcore_barrier()

    def gslice(j):
        return tbl.at[src_v.at[j]]

    for b in range(NBUF - 1):
        pltpu.async_copy(gslice(b), gb[b], sm[b])

    def grp(q, carry):
        for b in range(NBUF):
            j = NBUF * q + b
            pltpu.make_async_copy(gslice(j), gb[b], sm[b]).wait()
            jn = jnp.minimum(j + NBUF - 1, nch - 1)
            bn = (b + NBUF - 1) % NBUF
            pltpu.async_copy(gslice(jn), gb[bn], sm[bn])
            pltpu.sync_copy(gb[b], acc.at[dst_v.at[j]], add=True)
        return carry

    lax.fori_loop(0, nch // NBUF, grp, 0)
    for b in range(NBUF - 1):
        pltpu.make_async_copy(gslice(nch - 1), gb[b], sm[b]).wait()

    plsc.subcore_barrier()
    _flush_acc(fbuf, acc, out.at[cid], sid)


_out_prop_kernel = pl.kernel(
    functools.partial(_prop_staged_body, d=D_OUT, nch=NCH),
    out_type=jax.ShapeDtypeStruct((NC, N_PAD, D_OUT), jnp.float32),
    mesh=_MESH,
    scratch_types=[
        pltpu.VMEM((NCH, CHUNK), jnp.int32),
        pltpu.VMEM((NCH, CHUNK), jnp.int32),
    ]
    + [pltpu.VMEM((CHUNK, D_OUT), jnp.float32) for _ in range(NBUF)]
    + [
        pltpu.VMEM((FCHUNK, D_OUT), jnp.float32),
        pltpu.VMEM_SHARED((N_PAD, D_OUT), jnp.float32),
        pltpu.VMEM_SHARED((N_PAD, D_OUT), jnp.float32),
    ]
    + [pltpu.SemaphoreType.DMA for _ in range(NBUF)],
    compiler_params=pltpu.CompilerParams(use_tc_tiling_on_sc=False),
)


def _deg_body(dst_idx, out, dst_v, zbuf, obuf, acc):
    cid = lax.axis_index("c")
    sid = lax.axis_index("s")
    wid = cid * NS + sid

    _fill(zbuf, FCHUNK, D_DEG, 0.0)
    _fill(obuf, CHUNK, D_DEG, 1.0)
    _zero_acc(zbuf, acc, sid)
    pltpu.sync_copy(dst_idx.at[wid], dst_v)
    plsc.subcore_barrier()

    def step(j, carry):
        pltpu.sync_copy(obuf, acc.at[dst_v.at[j]], add=True)
        return carry

    lax.fori_loop(0, NCH, step, 0)

    plsc.subcore_barrier()
    _flush_acc(zbuf, acc, out.at[cid], sid)


_deg_kernel = pl.kernel(
    _deg_body,
    out_type=jax.ShapeDtypeStruct((NC, N_PAD, D_DEG), jnp.float32),
    mesh=_MESH,
    scratch_types=[
        pltpu.VMEM((NCH, CHUNK), jnp.int32),
        pltpu.VMEM((FCHUNK, D_DEG), jnp.float32),
        pltpu.VMEM((CHUNK, D_DEG), jnp.float32),
        pltpu.VMEM_SHARED((N_PAD, D_DEG), jnp.float32),
    ],
    compiler_params=pltpu.CompilerParams(use_tc_tiling_on_sc=False),
)


BM = 2000


def _dinv_of(da_ref, db_ref):
    return lax.rsqrt(da_ref[0, :, :1] + db_ref[0, :, :1] + 1.0)


def _mm1_body(x_ref, w_ref, da_ref, db_ref, o_ref):
    o_ref[...] = jnp.dot(x_ref[...], w_ref[...],
                         preferred_element_type=jnp.float32) * _dinv_of(da_ref, db_ref)


def _mid_body(aa_ref, ab_ref, s1_ref, da_ref, db_ref, b1_ref, w2_ref, o_ref):
    dinv = _dinv_of(da_ref, db_ref)
    agg = jnp.concatenate([aa_ref[0], ab_ref[0]], axis=1)
    z = (agg + s1_ref[...]) * dinv + b1_ref[...]
    h = jnp.maximum(z, 0.0)
    o_ref[...] = jnp.dot(h, w2_ref[...], preferred_element_type=jnp.float32) * dinv


def _final_body(aa_ref, ab_ref, s2_ref, da_ref, db_ref, b2_ref, o_ref):
    dinv = _dinv_of(da_ref, db_ref)
    z = (aa_ref[0] + ab_ref[0] + s2_ref[...]) * dinv + b2_ref[...]
    m = jnp.max(z, axis=1, keepdims=True)
    e = z - m
    o_ref[...] = e - jnp.log(jnp.sum(jnp.exp(e), axis=1, keepdims=True))


def _row_spec(d):
    return pl.BlockSpec((BM, d), lambda i: (i, 0))


def _plane_spec(d, c):
    return pl.BlockSpec((1, BM, d), lambda i, _c=c: (_c, i, 0))


def _deg_specs():
    return [_plane_spec(D_DEG, 0), _plane_spec(D_DEG, 1)]


def _mm1(x, w1, degs):
    return pl.pallas_call(
        _mm1_body,
        grid=(N // BM,),
        in_specs=[
            _row_spec(D_IN),
            pl.BlockSpec((D_IN, D_HID), lambda i: (0, 0)),
        ]
        + _deg_specs(),
        out_specs=_row_spec(D_HID),
        out_shape=jax.ShapeDtypeStruct((N, D_HID), jnp.float32),
    )(x, w1, degs, degs)


def _mid(agg1, scaled1, degs, b1, w2):
    return pl.pallas_call(
        _mid_body,
        grid=(N // BM,),
        in_specs=[
            _plane_spec(DH2, 0),
            _plane_spec(DH2, 1),
            _row_spec(D_HID),
        ]
        + _deg_specs()
        + [
            pl.BlockSpec((1, D_HID), lambda i: (0, 0)),
            pl.BlockSpec((D_HID, D_OUT), lambda i: (0, 0)),
        ],
        out_specs=_row_spec(D_OUT),
        out_shape=jax.ShapeDtypeStruct((N, D_OUT), jnp.float32),
    )(agg1, agg1, scaled1, degs, degs, b1, w2)


def _final(agg2, scaled2, degs, b2):
    return pl.pallas_call(
        _final_body,
        grid=(N // BM,),
        in_specs=[
            _plane_spec(D_OUT, 0),
            _plane_spec(D_OUT, 1),
            _row_spec(D_OUT),
        ]
        + _deg_specs()
        + [pl.BlockSpec((1, D_OUT), lambda i: (0, 0))],
        out_specs=_row_spec(D_OUT),
        out_shape=jax.ShapeDtypeStruct((N, D_OUT), jnp.float32),
    )(agg2, agg2, scaled2, degs, degs, b2)


def kernel(x, edge_index, W1, b1, W2, b2):
    ei = edge_index.astype(jnp.int32)
    src = ei[0].reshape(NW, NCH, CHUNK)
    dst = ei[1].reshape(NW, NCH, CHUNK)

    src16 = ei[0].reshape(NS, NCH2, CHUNK2) * 2
    src_cols = jnp.concatenate([src16, src16 + 1])
    dst16 = ei[1].reshape(NS, NCH2, CHUNK2)

    degs = _deg_kernel(dst)
    scaled1 = _mm1(x, W1, degs)
    table1 = scaled1.reshape(NC * N, DH2)
    agg1 = _hid_kernel(table1, src_cols, dst16)
    scaled2 = _mid(agg1, scaled1, degs,
                   b1.reshape(1, D_HID), W2)
    scaled2_pad = jnp.concatenate(
        [scaled2, jnp.zeros((N_PAD - N, D_OUT), jnp.float32)])
    agg2 = _out_prop_kernel(scaled2_pad, src, dst)
    return _final(agg2, scaled2, degs, b2.reshape(1, D_OUT))
